# baseline (speedup 1.0000x reference)
import jax
import jax.numpy as jnp
from jax.experimental import pallas as pl
from jax.experimental.pallas import tpu as pltpu

HIDDEN = 768
NUM_EXPERTS = 8
TOP_K = 2
NEG = -1e30


def _router_kernel(x_ref, w_ref, rw_ref, se_ref, aux_ref, acc_ref, *, n_tokens):
    i = pl.program_id(0)
    nblk = pl.num_programs(0)

    x = x_ref[...]
    w = w_ref[...]
    logits = jax.lax.dot_general(
        x, w, (((1,), (1,)), ((), ())),
        preferred_element_type=jnp.float32)

    iota = jax.lax.broadcasted_iota(jnp.int32, logits.shape, 1)
    m1 = jnp.max(logits, axis=1, keepdims=True)
    idx1 = jnp.min(jnp.where(logits == m1, iota, NUM_EXPERTS),
                   axis=1, keepdims=True)
    masked = jnp.where(iota == idx1, NEG, logits)
    m2 = jnp.max(masked, axis=1, keepdims=True)
    idx2 = jnp.min(jnp.where(masked == m2, iota, NUM_EXPERTS),
                   axis=1, keepdims=True)

    e2 = jnp.exp(m2 - m1)
    denom = 1.0 + e2
    rw_ref[...] = jnp.concatenate([1.0 / denom, e2 / denom], axis=1)
    se_ref[...] = jnp.concatenate([idx1, idx2], axis=1).astype(jnp.int32)

    ex = jnp.exp(logits - m1)
    probs = ex / jnp.sum(ex, axis=1, keepdims=True)
    psum = jnp.sum(probs, axis=0, keepdims=True)
    csum = jnp.sum((iota == idx1).astype(jnp.float32),
                   axis=0, keepdims=True)

    @pl.when(i == 0)
    def _init():
        acc_ref[...] = jnp.zeros_like(acc_ref)

    acc_ref[0:1, :] += psum
    acc_ref[1:2, :] += csum

    @pl.when(i == nblk - 1)
    def _final():
        p = acc_ref[0:1, :]
        c = acc_ref[1:2, :]
        scale = NUM_EXPERTS / float(n_tokens * n_tokens)
        aux_ref[...] = (scale * jnp.sum(p * c)).reshape(1, 1)


def kernel(x, W):
    B, S, H = x.shape
    n_tokens = B * S
    xf = x.reshape(n_tokens, H)
    BLK = 1024
    grid = (n_tokens // BLK,)

    import functools
    body = functools.partial(_router_kernel, n_tokens=n_tokens)

    rw, se, aux = pl.pallas_call(
        body,
        grid=grid,
        in_specs=[
            pl.BlockSpec((BLK, H), lambda i: (i, 0)),
            pl.BlockSpec((NUM_EXPERTS, H), lambda i: (0, 0)),
        ],
        out_specs=[
            pl.BlockSpec((BLK, TOP_K), lambda i: (i, 0)),
            pl.BlockSpec((BLK, TOP_K), lambda i: (i, 0)),
            pl.BlockSpec((1, 1), lambda i: (0, 0)),
        ],
        out_shape=[
            jax.ShapeDtypeStruct((n_tokens, TOP_K), jnp.float32),
            jax.ShapeDtypeStruct((n_tokens, TOP_K), jnp.int32),
            jax.ShapeDtypeStruct((1, 1), jnp.float32),
        ],
        scratch_shapes=[pltpu.VMEM((2, NUM_EXPERTS), jnp.float32)],
    )(xf, W)

    return (rw.reshape(B, S, TOP_K), se.reshape(B, S, TOP_K), aux[0, 0])


# speedup vs baseline: 1.9123x; 1.9123x over previous
import functools

import jax
import jax.numpy as jnp
from jax import lax
from jax.experimental import pallas as pl
from jax.experimental.pallas import tpu as pltpu

HIDDEN = 768
E = 8
TOP_K = 2
NEG = -1e30


def _router_kernel(x_ref, w_ref, rw_ref, se_ref, aux_ref, acc_ref, *, n_tokens):
    i = pl.program_id(0)
    nblk = pl.num_programs(0)

    x = x_ref[...]
    w = w_ref[...]
    logits = lax.dot_general(
        w, x, (((1,), (1,)), ((), ())),
        preferred_element_type=jnp.float32)

    iota = lax.broadcasted_iota(jnp.int32, logits.shape, 0)
    m1 = jnp.max(logits, axis=0, keepdims=True)
    i1 = jnp.min(jnp.where(logits == m1, iota, E), axis=0, keepdims=True)
    masked = jnp.where(iota == i1, NEG, logits)
    m2 = jnp.max(masked, axis=0, keepdims=True)
    i2 = jnp.min(jnp.where(masked == m2, iota, E), axis=0, keepdims=True)

    w1 = 1.0 / (1.0 + jnp.exp(m2 - m1))
    rw_ref[...] = jnp.concatenate([w1, 1.0 - w1], axis=0)
    se_ref[...] = jnp.concatenate([i1, i2], axis=0)

    ex = jnp.exp(logits - m1)
    probs = ex * (1.0 / jnp.sum(ex, axis=0, keepdims=True))
    psum = jnp.sum(probs, axis=1, keepdims=True)
    csum = jnp.sum((iota == i1).astype(jnp.float32),
                   axis=1, keepdims=True)

    @pl.when(i == 0)
    def _init():
        acc_ref[...] = jnp.zeros_like(acc_ref)

    acc_ref[:, 0:1] += psum
    acc_ref[:, 1:2] += csum

    @pl.when(i == nblk - 1)
    def _final():
        scale = E / float(n_tokens * n_tokens)
        aux_ref[...] = (scale * jnp.sum(acc_ref[:, 0:1] * acc_ref[:, 1:2])
                        ).reshape(1, 1)


def kernel(x, W):
    B, S, H = x.shape
    n_tokens = B * S
    xf = x.reshape(n_tokens, H)
    BLK = 1024
    grid = (n_tokens // BLK,)

    body = functools.partial(_router_kernel, n_tokens=n_tokens)

    rw_t, se_t, aux = pl.pallas_call(
        body,
        grid=grid,
        in_specs=[
            pl.BlockSpec((BLK, H), lambda i: (i, 0)),
            pl.BlockSpec((E, H), lambda i: (0, 0)),
        ],
        out_specs=[
            pl.BlockSpec((TOP_K, BLK), lambda i: (0, i)),
            pl.BlockSpec((TOP_K, BLK), lambda i: (0, i)),
            pl.BlockSpec((1, 1), lambda i: (0, 0)),
        ],
        out_shape=[
            jax.ShapeDtypeStruct((TOP_K, n_tokens), jnp.float32),
            jax.ShapeDtypeStruct((TOP_K, n_tokens), jnp.int32),
            jax.ShapeDtypeStruct((1, 1), jnp.float32),
        ],
        scratch_shapes=[pltpu.VMEM((E, 2), jnp.float32)],
    )(xf, W)

    rw = rw_t.T.reshape(B, S, TOP_K)
    se = se_t.T.reshape(B, S, TOP_K)
    return (rw, se, aux[0, 0])
